# baseline (speedup 1.0000x reference)
import functools

import jax
import jax.numpy as jnp
from jax import lax
from jax.experimental import pallas as pl
from jax.experimental.pallas import tpu as pltpu
from jax.experimental.pallas import tpu_sc as plsc

_FEAT = 128
_BATCH = 16384
_NC = 2
_NS = 16
_NW = _NC * _NS
_L = 16
_CH = _FEAT // _L

_ROWS = 128
_B_SC = _NW * _ROWS
_B_TC = _BATCH - _B_SC

_BLK = 64
_NBLK = _ROWS // _BLK
_GRP = _BLK // _L

_TBLK = 4096
_TGRID = _B_TC // _TBLK


def _sc_body(feat_hbm, label_hbm, wei_hbm, centers_hbm, out_hbm,
             fbuf, lbuf, wbuf, cbuf, obuf, sem0, sem1):
    wid = lax.axis_index("s") * _NC + lax.axis_index("c")
    base = wid * _ROWS
    sems = (sem0, sem1)
    copies = [None] * _NBLK

    def start(b):
        copies[b] = pltpu.async_copy(
            feat_hbm.at[pl.ds(base + b * _BLK, _BLK), :],
            fbuf.at[b % 2], sems[b % 2])

    start(0)
    if _NBLK > 1:
        start(1)
    pltpu.sync_copy(label_hbm.at[pl.ds(base, _ROWS)], lbuf)
    pltpu.sync_copy(wei_hbm.at[pl.ds(base, _ROWS)], wbuf)
    pltpu.sync_copy(centers_hbm, cbuf)

    zero = jnp.zeros((_L,), jnp.float32)
    carry = (
        tuple(zero for _ in range(_CH)),
        tuple(zero for _ in range(_CH)),
        tuple(zero for _ in range(_CH)),
        tuple(zero for _ in range(_CH)),
        zero, zero, zero,
    )

    for b in range(_NBLK):
        copies[b].wait()
        buf = fbuf.at[b % 2]

        def group_step(g, c, b=b, buf=buf):
            qv, pt, av, bv, wv_s, wav_s, wbv_s = c
            r0 = g * _L
            w16 = wbuf[pl.ds(b * _BLK + r0, _L)]
            lf16 = lbuf[pl.ds(b * _BLK + r0, _L)].astype(jnp.float32)
            a16 = w16 * lf16
            b16 = a16 * lf16
            qv = list(qv)
            pt = list(pt)
            av = list(av)
            bv = list(bv)
            for j in range(_L):
                w = w16[j]
                a = a16[j]
                bb = b16[j]
                for d in range(_CH):
                    f = buf[r0 + j, pl.ds(d * _L, _L)]
                    wf = w * f
                    qv[d] = qv[d] + wf * f
                    pt[d] = pt[d] + wf
                    av[d] = av[d] + a * f
                    bv[d] = bv[d] + bb * f
            return (tuple(qv), tuple(pt), tuple(av), tuple(bv),
                    wv_s + w16, wav_s + a16, wbv_s + b16)

        carry = lax.fori_loop(0, _GRP, group_step, carry)
        if b + 2 < _NBLK:
            start(b + 2)

    qv, pt, av, bv, wv_s, wav_s, wbv_s = carry
    w_s = jnp.sum(wv_s)
    wa_s = jnp.sum(wav_s)
    wb_s = jnp.sum(wbv_s)

    q16 = zero
    s1_16 = zero
    s2_16 = zero
    for d in range(_CH):
        c0 = cbuf[0, pl.ds(d * _L, _L)]
        c1 = cbuf[1, pl.ds(d * _L, _L)]
        c2 = cbuf[2, pl.ds(d * _L, _L)]
        u = c0
        v = 0.5 * (-3.0 * c0 + 4.0 * c1 - c2)
        z = 0.5 * (c0 - 2.0 * c1 + c2)
        q16 = q16 + qv[d]
        s1_16 = s1_16 + pt[d] * u + av[d] * v + bv[d] * z
        s2_16 = s2_16 + pt[d] * (c0 + c1 + c2)

    q_sc = jnp.sum(q16)
    s1_sc = jnp.sum(s1_16)
    s2_sc = jnp.sum(s2_16)

    lanes = lax.iota(jnp.int32, 16)
    outv = jnp.where(lanes == 0, q_sc, 0.0)
    outv = jnp.where(lanes == 1, s1_sc, outv)
    outv = jnp.where(lanes == 2, s2_sc, outv)
    outv = jnp.where(lanes == 3, w_s, outv)
    outv = jnp.where(lanes == 4, wa_s, outv)
    outv = jnp.where(lanes == 5, wb_s, outv)
    obuf[...] = outv
    pltpu.sync_copy(obuf, out_hbm.at[wid])


def _sc_partials(feat, label, wei, centers):
    mesh = plsc.VectorSubcoreMesh(core_axis_name="c", subcore_axis_name="s")
    return pl.kernel(
        _sc_body,
        out_type=jax.ShapeDtypeStruct((_NW, _L), jnp.float32),
        mesh=mesh,
        compiler_params=pltpu.CompilerParams(needs_layout_passes=False),
        scratch_types=[
            pltpu.VMEM((2, _BLK, _FEAT), jnp.float32),
            pltpu.VMEM((_ROWS,), jnp.int32),
            pltpu.VMEM((_ROWS,), jnp.float32),
            pltpu.VMEM((3, _FEAT), jnp.float32),
            pltpu.VMEM((_L,), jnp.float32),
            pltpu.SemaphoreType.DMA,
            pltpu.SemaphoreType.DMA,
        ],
    )(feat, label, wei, centers)


def _tc_body(feat_ref, lab_ref, wei_ref, cen_ref, sc_ref, out_ref, acc_ref):
    i = pl.program_id(0)

    @pl.when(i == 0)
    def _():
        acc_ref[...] = jnp.zeros((8, _FEAT), jnp.float32)

    lf = lab_ref[0, 0, :].astype(jnp.float32)
    w = wei_ref[0, 0, :]
    a = w * lf
    b = a * lf
    wab = jnp.concatenate([w[None, :], a[None, :], b[None, :]], axis=0)
    f = feat_ref[...]
    pab = jnp.dot(wab, f, preferred_element_type=jnp.float32)
    qv = jnp.dot(w[None, :], f * f,
                 preferred_element_type=jnp.float32)
    lane1 = lax.broadcasted_iota(jnp.int32, (1, _FEAT), 1)
    wrow = jnp.where(lane1 == 0, jnp.sum(w), 0.0)
    wrow = jnp.where(lane1 == 1, jnp.sum(a), wrow)
    wrow = jnp.where(lane1 == 2, jnp.sum(b), wrow)
    upd = jnp.concatenate(
        [pab, qv, wrow, jnp.zeros((3, _FEAT), jnp.float32)], axis=0)
    acc_ref[...] += upd

    @pl.when(i == _TGRID - 1)
    def _():
        acc = acc_ref[...]
        c0 = cen_ref[0, :]
        c1 = cen_ref[1, :]
        c2 = cen_ref[2, :]
        u = c0
        v = 0.5 * (-3.0 * c0 + 4.0 * c1 - c2)
        z = 0.5 * (c0 - 2.0 * c1 + c2)

        t2 = jnp.sum(sc_ref[...], axis=0, keepdims=True)
        lane16 = lax.broadcasted_iota(jnp.int32, (1, 16), 1)

        def sc_lane(k):
            return jnp.sum(jnp.where(lane16 == k, t2, 0.0))

        wrow_acc = acc[4:5, :]

        def w_lane(k):
            return jnp.sum(jnp.where(lane1 == k, wrow_acc, 0.0))

        q = sc_lane(0) + jnp.sum(acc[3, :])
        s1 = sc_lane(1) + jnp.sum(acc[0, :] * u + acc[1, :] * v
                                  + acc[2, :] * z)
        s2 = sc_lane(2) + jnp.sum(acc[0, :] * (c0 + c1 + c2))
        w_s = sc_lane(3) + w_lane(0)
        wa_s = sc_lane(4) + w_lane(1)
        wb_s = sc_lane(5) + w_lane(2)

        ck0 = jnp.sum(c0 * c0)
        ck1 = jnp.sum(c1 * c1)
        ck2_ = jnp.sum(c2 * c2)
        t_own = (w_s * ck0
                 + wa_s * 0.5 * (-3.0 * ck0 + 4.0 * ck1 - ck2_)
                 + wb_s * 0.5 * (ck0 - 2.0 * ck1 + ck2_))
        s_own = q - 2.0 * s1 + t_own
        s_all = 3.0 * q - 2.0 * s2 + w_s * (ck0 + ck1 + ck2_)
        distocen = s_all - s_own
        loss = s_own * (1.0 + 1.0 / distocen) / 2.0 / _BATCH
        out_ref[0] = loss


def _tc_final(feat, label, wei, centers, sc_part):
    lab2 = label.reshape(-1, 1, _TBLK)
    wei2 = wei.reshape(-1, 1, _TBLK)
    off = _B_SC // _TBLK
    return pl.pallas_call(
        _tc_body,
        grid=(_TGRID,),
        in_specs=[
            pl.BlockSpec((_TBLK, _FEAT), lambda i: (i + off, 0)),
            pl.BlockSpec((1, 1, _TBLK), lambda i: (i + off, 0, 0)),
            pl.BlockSpec((1, 1, _TBLK), lambda i: (i + off, 0, 0)),
            pl.BlockSpec((3, _FEAT), lambda i: (0, 0)),
            pl.BlockSpec((_NW, _L), lambda i: (0, 0)),
        ],
        out_specs=pl.BlockSpec(memory_space=pltpu.SMEM),
        out_shape=jax.ShapeDtypeStruct((1,), jnp.float32),
        scratch_shapes=[pltpu.VMEM((8, _FEAT), jnp.float32)],
        compiler_params=pltpu.CompilerParams(
            dimension_semantics=("arbitrary",)),
    )(feat, lab2, wei2, centers, sc_part)


@jax.jit
def _loss(feat, label, wei, centers):
    sc_part = _sc_partials(feat, label, wei, centers)
    out = _tc_final(feat, label, wei, centers, sc_part)
    return out[0]


def kernel(feat, label, wei, centers):
    return _loss(feat, label.astype(jnp.int32), wei, centers)


# speedup vs baseline: 1.0275x; 1.0275x over previous
import jax
import jax.numpy as jnp
from jax import lax
from jax.experimental import pallas as pl
from jax.experimental.pallas import tpu as pltpu
from jax.experimental.pallas import tpu_sc as plsc

_FEAT = 128
_BATCH = 16384
_NC = 2
_NS = 16
_NW = _NC * _NS
_L = 16
_CH = _FEAT // _L

_ROWS = 128
_B_SC = _NW * _ROWS
_B_TC = _BATCH - _B_SC

_BLK = 128
_NBLK = _ROWS // _BLK
_GRP = _BLK // _L

_TBLK = 4096
_TGRID = _B_TC // _TBLK


def _sc_body(feat_hbm, label_hbm, wei_hbm, centers_hbm, out_hbm,
             fbuf, lbuf, wbuf, cbuf, obuf, sem0, sem1):
    wid = lax.axis_index("s") * _NC + lax.axis_index("c")
    base = wid * _ROWS
    sems = (sem0, sem1)
    copies = [None] * _NBLK

    def start(b):
        copies[b] = pltpu.async_copy(
            feat_hbm.at[pl.ds(base + b * _BLK, _BLK), :],
            fbuf.at[b % 2], sems[b % 2])

    start(0)
    if _NBLK > 1:
        start(1)
    pltpu.sync_copy(label_hbm.at[pl.ds(base, _ROWS)], lbuf)
    pltpu.sync_copy(wei_hbm.at[pl.ds(base, _ROWS)], wbuf)
    pltpu.sync_copy(centers_hbm, cbuf)

    zero = jnp.zeros((_L,), jnp.float32)
    carry = (
        tuple(zero for _ in range(_CH)),
        tuple(zero for _ in range(_CH)),
        tuple(zero for _ in range(_CH)),
        tuple(zero for _ in range(_CH)),
        zero, zero, zero,
    )

    for b in range(_NBLK):
        copies[b].wait()
        buf = fbuf.at[b % 2]

        def group_step(g, c, b=b, buf=buf):
            qv, pt, av, bv, wv_s, wav_s, wbv_s = c
            r0 = g * _L
            w16 = wbuf[pl.ds(b * _BLK + r0, _L)]
            lf16 = lbuf[pl.ds(b * _BLK + r0, _L)].astype(jnp.float32)
            a16 = w16 * lf16
            b16 = a16 * lf16
            qv = list(qv)
            pt = list(pt)
            av = list(av)
            bv = list(bv)
            for j in range(_L):
                w = w16[j]
                a = a16[j]
                bb = b16[j]
                for d in range(_CH):
                    f = buf[r0 + j, pl.ds(d * _L, _L)]
                    wf = w * f
                    qv[d] = qv[d] + wf * f
                    pt[d] = pt[d] + wf
                    av[d] = av[d] + a * f
                    bv[d] = bv[d] + bb * f
            return (tuple(qv), tuple(pt), tuple(av), tuple(bv),
                    wv_s + w16, wav_s + a16, wbv_s + b16)

        carry = lax.fori_loop(0, _GRP, group_step, carry)
        if b + 2 < _NBLK:
            start(b + 2)

    qv, pt, av, bv, wv_s, wav_s, wbv_s = carry
    w_s = jnp.sum(wv_s)
    wa_s = jnp.sum(wav_s)
    wb_s = jnp.sum(wbv_s)

    q16 = zero
    s1_16 = zero
    s2_16 = zero
    for d in range(_CH):
        c0 = cbuf[0, pl.ds(d * _L, _L)]
        c1 = cbuf[1, pl.ds(d * _L, _L)]
        c2 = cbuf[2, pl.ds(d * _L, _L)]
        u = c0
        v = 0.5 * (-3.0 * c0 + 4.0 * c1 - c2)
        z = 0.5 * (c0 - 2.0 * c1 + c2)
        q16 = q16 + qv[d]
        s1_16 = s1_16 + pt[d] * u + av[d] * v + bv[d] * z
        s2_16 = s2_16 + pt[d] * (c0 + c1 + c2)

    q_sc = jnp.sum(q16)
    s1_sc = jnp.sum(s1_16)
    s2_sc = jnp.sum(s2_16)

    lanes = lax.iota(jnp.int32, 16)
    outv = jnp.where(lanes == 0, q_sc, 0.0)
    outv = jnp.where(lanes == 1, s1_sc, outv)
    outv = jnp.where(lanes == 2, s2_sc, outv)
    outv = jnp.where(lanes == 3, w_s, outv)
    outv = jnp.where(lanes == 4, wa_s, outv)
    outv = jnp.where(lanes == 5, wb_s, outv)
    obuf[...] = outv
    pltpu.sync_copy(obuf, out_hbm.at[wid])


def _sc_partials(feat, label, wei, centers):
    mesh = plsc.VectorSubcoreMesh(core_axis_name="c", subcore_axis_name="s")
    return pl.kernel(
        _sc_body,
        out_type=jax.ShapeDtypeStruct((_NW, _L), jnp.float32),
        mesh=mesh,
        compiler_params=pltpu.CompilerParams(needs_layout_passes=False),
        scratch_types=[
            pltpu.VMEM((2, _BLK, _FEAT), jnp.float32),
            pltpu.VMEM((_ROWS,), jnp.int32),
            pltpu.VMEM((_ROWS,), jnp.float32),
            pltpu.VMEM((3, _FEAT), jnp.float32),
            pltpu.VMEM((_L,), jnp.float32),
            pltpu.SemaphoreType.DMA,
            pltpu.SemaphoreType.DMA,
        ],
    )(feat, label, wei, centers)


def _tc_body(feat_ref, lab_ref, wei_ref, cen_ref, sc_ref, out_ref, acc_ref):
    i = pl.program_id(0)

    @pl.when(i == 0)
    def _():
        acc_ref[...] = jnp.zeros((8, _FEAT), jnp.float32)

    lf = lab_ref[0, 0, :].astype(jnp.float32)
    w = wei_ref[0, 0, :]
    a = w * lf
    b = a * lf
    wab = jnp.concatenate([w[None, :], a[None, :], b[None, :]], axis=0)
    f = feat_ref[...]
    pab = jnp.dot(wab, f, preferred_element_type=jnp.float32)
    qv = jnp.dot(w[None, :], f * f,
                 preferred_element_type=jnp.float32)
    lane1 = lax.broadcasted_iota(jnp.int32, (1, _FEAT), 1)
    wrow = jnp.where(lane1 == 0, jnp.sum(w), 0.0)
    wrow = jnp.where(lane1 == 1, jnp.sum(a), wrow)
    wrow = jnp.where(lane1 == 2, jnp.sum(b), wrow)
    upd = jnp.concatenate(
        [pab, qv, wrow, jnp.zeros((3, _FEAT), jnp.float32)], axis=0)
    acc_ref[...] += upd

    @pl.when(i == _TGRID - 1)
    def _():
        acc = acc_ref[...]
        c0 = cen_ref[0, :]
        c1 = cen_ref[1, :]
        c2 = cen_ref[2, :]
        u = c0
        v = 0.5 * (-3.0 * c0 + 4.0 * c1 - c2)
        z = 0.5 * (c0 - 2.0 * c1 + c2)

        t2 = jnp.sum(sc_ref[...], axis=0, keepdims=True)
        lane16 = lax.broadcasted_iota(jnp.int32, (1, 16), 1)

        def sc_lane(k):
            return jnp.sum(jnp.where(lane16 == k, t2, 0.0))

        wrow_acc = acc[4:5, :]

        def w_lane(k):
            return jnp.sum(jnp.where(lane1 == k, wrow_acc, 0.0))

        q = sc_lane(0) + jnp.sum(acc[3, :])
        s1 = sc_lane(1) + jnp.sum(acc[0, :] * u + acc[1, :] * v
                                  + acc[2, :] * z)
        s2 = sc_lane(2) + jnp.sum(acc[0, :] * (c0 + c1 + c2))
        w_s = sc_lane(3) + w_lane(0)
        wa_s = sc_lane(4) + w_lane(1)
        wb_s = sc_lane(5) + w_lane(2)

        ck0 = jnp.sum(c0 * c0)
        ck1 = jnp.sum(c1 * c1)
        ck2_ = jnp.sum(c2 * c2)
        t_own = (w_s * ck0
                 + wa_s * 0.5 * (-3.0 * ck0 + 4.0 * ck1 - ck2_)
                 + wb_s * 0.5 * (ck0 - 2.0 * ck1 + ck2_))
        s_own = q - 2.0 * s1 + t_own
        s_all = 3.0 * q - 2.0 * s2 + w_s * (ck0 + ck1 + ck2_)
        distocen = s_all - s_own
        loss = s_own * (1.0 + 1.0 / distocen) / 2.0 / _BATCH
        out_ref[0] = loss


def _tc_final(feat, label, wei, centers, sc_part):
    lab2 = label.reshape(-1, 1, _TBLK)
    wei2 = wei.reshape(-1, 1, _TBLK)
    off = _B_SC // _TBLK
    return pl.pallas_call(
        _tc_body,
        grid=(_TGRID,),
        in_specs=[
            pl.BlockSpec((_TBLK, _FEAT), lambda i: (i + off, 0)),
            pl.BlockSpec((1, 1, _TBLK), lambda i: (i + off, 0, 0)),
            pl.BlockSpec((1, 1, _TBLK), lambda i: (i + off, 0, 0)),
            pl.BlockSpec((3, _FEAT), lambda i: (0, 0)),
            pl.BlockSpec((_NW, _L), lambda i: (0, 0)),
        ],
        out_specs=pl.BlockSpec(memory_space=pltpu.SMEM),
        out_shape=jax.ShapeDtypeStruct((1,), jnp.float32),
        scratch_shapes=[pltpu.VMEM((8, _FEAT), jnp.float32)],
        compiler_params=pltpu.CompilerParams(
            dimension_semantics=("arbitrary",)),
    )(feat, lab2, wei2, centers, sc_part)


@jax.jit
def _loss(feat, label, wei, centers):
    sc_part = _sc_partials(feat, label, wei, centers)
    out = _tc_final(feat, label, wei, centers, sc_part)
    return out[0]


def kernel(feat, label, wei, centers):
    return _loss(feat, label.astype(jnp.int32), wei, centers)
